# Initial kernel scaffold; baseline (speedup 1.0000x reference)
import jax
import jax.numpy as jnp
from jax.experimental import pallas as pl


def kernel(node_attr, edge_index, edge_attr, edge_feat, fc_w1, fc_b1, fc_w2, fc_b2):
    raise NotImplementedError("write your pallas kernel here")


# speedup vs baseline: 1.8522x; 1.8522x over previous
import functools

import numpy as np
import jax
import jax.numpy as jnp
from jax import lax
from jax.experimental import pallas as pl
from jax.experimental.pallas import tpu as pltpu

_NS = 16
_NV = 8
_NE = 160000
_NN = 10000
_NJ = 960
_ALPHA = 1.0 / np.sqrt(24.0)


@functools.cache
def _build_consts():
    E = np.zeros((576, _NJ), np.float32)
    C = np.zeros((_NJ, 40), np.float32)
    A0 = np.zeros((16, 256), np.float32)
    M = np.zeros((24, 128), np.float32)
    A1 = np.zeros((16, 384), np.float32)
    B1 = np.zeros((3, 384), np.float32)
    A2 = np.zeros((24, 192), np.float32)
    Tb = np.zeros((3, 24), np.float32)
    s3 = 1.0 / np.sqrt(3.0)
    for u in range(16):
        for w in range(16):
            j = u * 16 + w
            E[u * 16 + w, j] = 1.0
            C[j, w] = _ALPHA
            A0[u, j] = 1.0
    for u in range(8):
        for w in range(16):
            j = 256 + u * 16 + w
            E[448 + u * 16 + w, j] = 1.0
            C[j, w] = _ALPHA * s3
        for i in range(3):
            Tb[i, u * 3 + i] = 1.0
            for w in range(16):
                M[u * 3 + i, u * 16 + w] = 1.0
    for u in range(16):
        for w in range(8):
            for i in range(3):
                j = 384 + (u * 8 + w) * 3 + i
                E[256 + u * 8 + w, j] = 1.0
                C[j, 16 + w * 3 + i] = _ALPHA
                A1[u, j - 384] = 1.0
                B1[i, j - 384] = 1.0
    for u in range(8):
        for w in range(8):
            for i in range(3):
                j = 768 + (u * 8 + w) * 3 + i
                E[384 + u * 8 + w, j] = 1.0
                C[j, 16 + w * 3 + i] = _ALPHA
                A2[u * 3 + i, j - 768] = 1.0
    return E, C, A0, M, A1, B1, A2, Tb


def _tp_body(ef_ref, x_ref, sh_ref, w1_ref, b1_ref, w2e_ref, b2e_ref,
             a0_ref, m_ref, a1_ref, b1m_ref, a2_ref, tb_ref, c_ref, o_ref):
    f32 = jnp.float32
    h = jnp.maximum(
        jnp.dot(ef_ref[...], w1_ref[...], preferred_element_type=f32)
        + b1_ref[...], 0.0)
    wexp = jnp.dot(h, w2e_ref[...], preferred_element_type=f32) + b2e_ref[...]
    x = x_ref[...]
    s_in = x[:, :_NS]
    xv = x[:, _NS:]
    sh = sh_ref[...]
    sh0 = sh[:, 0:1]
    sh1 = sh[:, 1:4]
    z000 = jnp.dot(s_in, a0_ref[...], preferred_element_type=f32) * sh0
    sh1t = jnp.dot(sh1, tb_ref[...], preferred_element_type=f32)
    z110 = jnp.dot(xv * sh1t, m_ref[...], preferred_element_type=f32)
    z011 = (jnp.dot(s_in, a1_ref[...], preferred_element_type=f32)
            * jnp.dot(sh1, b1m_ref[...], preferred_element_type=f32))
    z101 = jnp.dot(xv, a2_ref[...], preferred_element_type=f32) * sh0
    z = jnp.concatenate([z000, z110, z011, z101], axis=1)
    o_ref[...] = jnp.dot(wexp * z, c_ref[...], preferred_element_type=f32)


def _tc_tensor_product(edge_feat, x_dst, edge_attr, fc_w1, fc_b1, w2e, b2e,
                       interpret=False):
    E_np, C, A0, M, A1, B1, A2, Tb = _build_consts()
    del E_np
    T = 2000
    grid = (_NE // T,)
    row = lambda i: (i, 0)
    rep = lambda i: (0, 0)
    consts = [jnp.asarray(a) for a in (A0, M, A1, B1, A2, Tb, C)]
    in_specs = (
        [pl.BlockSpec((T, 16), row),
         pl.BlockSpec((T, 40), row),
         pl.BlockSpec((T, 4), row),
         pl.BlockSpec((16, 128), rep),
         pl.BlockSpec((1, 128), rep),
         pl.BlockSpec((128, _NJ), rep),
         pl.BlockSpec((1, _NJ), rep)]
        + [pl.BlockSpec(c.shape, rep) for c in consts]
    )
    return pl.pallas_call(
        _tp_body,
        grid=grid,
        in_specs=in_specs,
        out_specs=pl.BlockSpec((T, 40), row),
        out_shape=jax.ShapeDtypeStruct((_NE, 40), jnp.float32),
        interpret=interpret,
    )(edge_feat, x_dst, edge_attr, fc_w1, fc_b1[None, :], w2e, b2e[None, :],
      *consts)


def kernel(node_attr, edge_index, edge_attr, edge_feat,
           fc_w1, fc_b1, fc_w2, fc_b2):
    E_np = _build_consts()[0]
    Ej = jnp.asarray(E_np)
    w2e = fc_w2 @ Ej
    b2e = fc_b2 @ Ej
    dst = edge_index[1]
    src = edge_index[0]
    x_dst = jnp.take(node_attr, dst, axis=0)
    tp = _tc_tensor_product(edge_feat, x_dst, edge_attr, fc_w1, fc_b1, w2e, b2e)
    out = jax.ops.segment_sum(tp, src, num_segments=_NN)
    return out


# speedup vs baseline: 4.0963x; 2.2116x over previous
import functools

import numpy as np
import jax
import jax.numpy as jnp
from jax import lax
from jax.experimental import pallas as pl
from jax.experimental.pallas import tpu as pltpu
from jax.experimental.pallas import tpu_sc as plsc

_NS = 16
_NV = 8
_NE = 160000
_NN = 10000
_NJ = 960
_ALPHA = 1.0 / np.sqrt(24.0)


@functools.cache
def _build_consts():
    E = np.zeros((576, _NJ), np.float32)
    C = np.zeros((_NJ, 40), np.float32)
    A0 = np.zeros((16, 256), np.float32)
    M = np.zeros((24, 128), np.float32)
    A1 = np.zeros((16, 384), np.float32)
    B1 = np.zeros((3, 384), np.float32)
    A2 = np.zeros((24, 192), np.float32)
    Tb = np.zeros((3, 24), np.float32)
    s3 = 1.0 / np.sqrt(3.0)
    for u in range(16):
        for w in range(16):
            j = u * 16 + w
            E[u * 16 + w, j] = 1.0
            C[j, w] = _ALPHA
            A0[u, j] = 1.0
    for u in range(8):
        for w in range(16):
            j = 256 + u * 16 + w
            E[448 + u * 16 + w, j] = 1.0
            C[j, w] = _ALPHA * s3
        for i in range(3):
            Tb[i, u * 3 + i] = 1.0
            for w in range(16):
                M[u * 3 + i, u * 16 + w] = 1.0
    for u in range(16):
        for w in range(8):
            for i in range(3):
                j = 384 + (u * 8 + w) * 3 + i
                E[256 + u * 8 + w, j] = 1.0
                C[j, 16 + w * 3 + i] = _ALPHA
                A1[u, j - 384] = 1.0
                B1[i, j - 384] = 1.0
    for u in range(8):
        for w in range(8):
            for i in range(3):
                j = 768 + (u * 8 + w) * 3 + i
                E[384 + u * 8 + w, j] = 1.0
                C[j, 16 + w * 3 + i] = _ALPHA
                A2[u * 3 + i, j - 768] = 1.0
    return E, C, A0, M, A1, B1, A2, Tb


def _tp_body(ef_ref, x_ref, sh_ref, w1_ref, b1_ref, w2e_ref, b2e_ref,
             a0_ref, m_ref, a1_ref, b1m_ref, a2_ref, tb_ref, c_ref, o_ref):
    f32 = jnp.float32
    h = jnp.maximum(
        jnp.dot(ef_ref[...], w1_ref[...], preferred_element_type=f32)
        + b1_ref[...], 0.0)
    wexp = jnp.dot(h.astype(jnp.bfloat16), w2e_ref[...],
                   preferred_element_type=f32) + b2e_ref[...]
    x = x_ref[...]
    s_in = x[:, :_NS]
    xv = x[:, _NS:40]
    sh = sh_ref[...]
    sh0 = sh[:, 0:1]
    sh1 = sh[:, 1:4]
    z000 = jnp.dot(s_in, a0_ref[...], preferred_element_type=f32) * sh0
    sh1t = jnp.dot(sh1, tb_ref[...], preferred_element_type=f32)
    z110 = jnp.dot(xv * sh1t, m_ref[...], preferred_element_type=f32)
    z011 = (jnp.dot(s_in, a1_ref[...], preferred_element_type=f32)
            * jnp.dot(sh1, b1m_ref[...], preferred_element_type=f32))
    z101 = jnp.dot(xv, a2_ref[...], preferred_element_type=f32) * sh0
    z = jnp.concatenate([z000, z110, z011, z101], axis=1)
    tp = jnp.dot((wexp * z).astype(jnp.bfloat16), c_ref[...],
                 preferred_element_type=f32)
    o_ref[...] = jnp.concatenate(
        [tp, jnp.zeros((tp.shape[0], 88), f32)], axis=1)


def _tc_tensor_product(edge_feat, x_dst, edge_attr, fc_w1, fc_b1, w2e, b2e,
                       interpret=False):
    E_np, C, A0, M, A1, B1, A2, Tb = _build_consts()
    del E_np
    T = 2000
    grid = (_NE // T,)
    row = lambda i: (i, 0)
    rep = lambda i: (0, 0)
    consts = [jnp.asarray(a) for a in (A0, M, A1, B1, A2, Tb)]
    consts.append(jnp.asarray(C, jnp.bfloat16))
    in_specs = (
        [pl.BlockSpec((T, 16), row),
         pl.BlockSpec((T, 128), row),
         pl.BlockSpec((T, 4), row),
         pl.BlockSpec((16, 128), rep),
         pl.BlockSpec((1, 128), rep),
         pl.BlockSpec((128, _NJ), rep),
         pl.BlockSpec((1, _NJ), rep)]
        + [pl.BlockSpec(c.shape, rep) for c in consts]
    )
    return pl.pallas_call(
        _tp_body,
        grid=grid,
        in_specs=in_specs,
        out_specs=pl.BlockSpec((T, 128), row),
        out_shape=jax.ShapeDtypeStruct((_NE, 128), jnp.float32),
        interpret=interpret,
    )(edge_feat, x_dst, edge_attr, fc_w1, fc_b1[None, :], w2e, b2e[None, :],
      *consts)


_NC = 2
_NSUB = 16
_NW = _NC * _NSUB
_BPW = _NE // _NW
_CHG = 1000
_NCHG = _BPW // _CHG
_CHS = 200
_NCHS = _BPW // _CHS


def _sc_gather(node_attr, dst):
    mesh = plsc.VectorSubcoreMesh(core_axis_name="c", subcore_axis_name="s")

    @functools.partial(
        pl.kernel,
        out_type=jax.ShapeDtypeStruct((_NE, 128), jnp.float32),
        mesh=mesh,
        scratch_types=[
            pltpu.VMEM((_CHG,), jnp.int32),
            pltpu.VMEM((_CHG, 128), jnp.float32),
            pltpu.SemaphoreType.DMA,
        ],
    )
    def k(table_hbm, idx_hbm, out_hbm, idx_v, rows_v, sem):
        wid = lax.axis_index("s") * _NC + lax.axis_index("c")
        base = wid * _BPW
        for c in range(_NCHG):
            off = base + c * _CHG
            pltpu.sync_copy(idx_hbm.at[pl.ds(off, _CHG)], idx_v)
            pltpu.async_copy(table_hbm.at[idx_v], rows_v, sem).wait()
            pltpu.sync_copy(rows_v, out_hbm.at[pl.ds(off, _CHG)])

    return k(node_attr, dst)


_SCB = 128
_EPT = _NE // _NSUB
_NSCB = _EPT // _SCB
_STAIL = _EPT - _NSCB * _SCB
_HALF = _NN // _NC
_ACCR = _HALF + 120


def _sc_scatter(tp128, src, zeros):
    mesh = plsc.VectorSubcoreMesh(core_axis_name="c", subcore_axis_name="s")

    @functools.partial(
        pl.kernel,
        out_type=jax.ShapeDtypeStruct((_NN, 128), jnp.float32),
        mesh=mesh,
        scratch_types=[
            pltpu.VMEM((_SCB,), jnp.int32),
            pltpu.VMEM((_SCB,), jnp.int32),
            pltpu.VMEM((_SCB, 128), jnp.float32),
            pltpu.VMEM((_SCB,), jnp.int32),
            pltpu.VMEM((_SCB,), jnp.int32),
            pltpu.VMEM((_SCB, 128), jnp.float32),
            pltpu.VMEM((_STAIL,), jnp.int32),
            pltpu.VMEM((_STAIL,), jnp.int32),
            pltpu.VMEM((_STAIL, 128), jnp.float32),
            pltpu.VMEM_SHARED((_ACCR, 128), jnp.float32),
            pltpu.SemaphoreType.DMA,
            pltpu.SemaphoreType.DMA,
        ],
    )
    def k(tp_hbm, src_hbm, z_hbm, out_hbm,
          idx_v, lidx_v, rows_v, idx_v2, lidx_v2, rows_v2,
          idx_t, lidx_t, rows_t, acc_sh, sem, sem2):
        cid = lax.axis_index("c")
        sid = lax.axis_index("s")
        lo = cid * _HALF
        zpt = _ACCR // _NSUB
        pltpu.sync_copy(z_hbm.at[pl.ds(sid * zpt, zpt)],
                        acc_sh.at[pl.ds(sid * zpt, zpt)])
        plsc.subcore_barrier()

        def remap(n, src_idx, dst_idx):
            for g in range(n // 16):
                v = src_idx[pl.ds(g * 16, 16)]
                lv = v - lo
                ok = (lv >= 0) & (lv < _HALF)
                dst_idx[pl.ds(g * 16, 16)] = jnp.where(ok, lv, _HALF)

        base = sid * _EPT
        def fetch(j, b):
            iv, rv, sm = (idx_v, rows_v, sem) if b == 0 else (idx_v2, rows_v2, sem2)
            off = base + j * _SCB
            pltpu.async_copy(src_hbm.at[pl.ds(off, _SCB)], iv, sm)
            pltpu.async_copy(tp_hbm.at[pl.ds(off, _SCB)], rv, sm)

        def drain(b):
            iv, rv, sm = (idx_v, rows_v, sem) if b == 0 else (idx_v2, rows_v2, sem2)
            pltpu.make_async_copy(src_hbm.at[pl.ds(0, _SCB)], iv, sm).wait()
            pltpu.make_async_copy(tp_hbm.at[pl.ds(0, _SCB)], rv, sm).wait()

        def scat(b):
            iv, rv, _ = (idx_v, rows_v, sem) if b == 0 else (idx_v2, rows_v2, sem2)
            li = lidx_v if b == 0 else lidx_v2
            remap(_SCB, iv, li)
            pltpu.sync_copy(rv, acc_sh.at[li], add=True)

        fetch(0, 0)
        def body(i, carry):
            fetch(2 * i + 1, 1)
            drain(0)
            scat(0)
            @pl.when(i < _NSCB // 2 - 1)
            def _():
                fetch(2 * i + 2, 0)
            drain(1)
            scat(1)
            return carry
        lax.fori_loop(0, _NSCB // 2, body, 0)
        offt = base + _NSCB * _SCB
        pltpu.sync_copy(src_hbm.at[pl.ds(offt, _STAIL)], idx_t)
        pltpu.sync_copy(tp_hbm.at[pl.ds(offt, _STAIL)], rows_t)
        remap(_STAIL, idx_t, lidx_t)
        pltpu.sync_copy(rows_t, acc_sh.at[lidx_t], add=True)
        plsc.subcore_barrier()
        @pl.when(sid < 5)
        def _():
            pltpu.sync_copy(acc_sh.at[pl.ds(sid * 1000, 1000)],
                            out_hbm.at[pl.ds(cid * _HALF + sid * 1000, 1000)])

    return k(tp128, src, zeros)


def kernel(node_attr, edge_index, edge_attr, edge_feat,
           fc_w1, fc_b1, fc_w2, fc_b2):
    E_np = _build_consts()[0]
    Ej = jnp.asarray(E_np)
    w2e = (fc_w2 @ Ej).astype(jnp.bfloat16)
    b2e = fc_b2 @ Ej
    dst = edge_index[1]
    src = edge_index[0]
    node_pad = jnp.pad(node_attr, ((0, 0), (0, 128 - 40)))
    x_dst = _sc_gather(node_pad, dst)
    tp = _tc_tensor_product(edge_feat, x_dst, edge_attr, fc_w1, fc_b1, w2e, b2e)
    zeros = jnp.zeros((_ACCR, 128), jnp.float32)
    out128 = _sc_scatter(tp, src, zeros)
    return out128[:, :40]


# speedup vs baseline: 4.3513x; 1.0622x over previous
import functools

import numpy as np
import jax
import jax.numpy as jnp
from jax import lax
from jax.experimental import pallas as pl
from jax.experimental.pallas import tpu as pltpu
from jax.experimental.pallas import tpu_sc as plsc

_NS = 16
_NV = 8
_NE = 160000
_NN = 10000
_NJ = 960
_ALPHA = 1.0 / np.sqrt(24.0)


@functools.cache
def _build_consts():
    E = np.zeros((576, _NJ), np.float32)
    C = np.zeros((_NJ, 40), np.float32)
    A0 = np.zeros((16, 256), np.float32)
    M = np.zeros((24, 128), np.float32)
    A1 = np.zeros((16, 384), np.float32)
    B1 = np.zeros((3, 384), np.float32)
    A2 = np.zeros((24, 192), np.float32)
    Tb = np.zeros((3, 24), np.float32)
    s3 = 1.0 / np.sqrt(3.0)
    for u in range(16):
        for w in range(16):
            j = u * 16 + w
            E[u * 16 + w, j] = 1.0
            C[j, w] = _ALPHA
            A0[u, j] = 1.0
    for u in range(8):
        for w in range(16):
            j = 256 + u * 16 + w
            E[448 + u * 16 + w, j] = 1.0
            C[j, w] = _ALPHA * s3
        for i in range(3):
            Tb[i, u * 3 + i] = 1.0
            for w in range(16):
                M[u * 3 + i, u * 16 + w] = 1.0
    for u in range(16):
        for w in range(8):
            for i in range(3):
                j = 384 + (u * 8 + w) * 3 + i
                E[256 + u * 8 + w, j] = 1.0
                C[j, 16 + w * 3 + i] = _ALPHA
                A1[u, j - 384] = 1.0
                B1[i, j - 384] = 1.0
    for u in range(8):
        for w in range(8):
            for i in range(3):
                j = 768 + (u * 8 + w) * 3 + i
                E[384 + u * 8 + w, j] = 1.0
                C[j, 16 + w * 3 + i] = _ALPHA
                A2[u * 3 + i, j - 768] = 1.0
    return E, C, A0, M, A1, B1, A2, Tb


def _tp_body(ef_ref, x_ref, sh_ref, w1_ref, b1_ref, w2e_ref, b2e_ref,
             a0_ref, m_ref, a1_ref, b1m_ref, a2_ref, tb_ref, c_ref, o_ref):
    f32 = jnp.float32
    h = jnp.maximum(
        jnp.dot(ef_ref[...], w1_ref[...], preferred_element_type=f32)
        + b1_ref[...], 0.0)
    wexp = jnp.dot(h.astype(jnp.bfloat16), w2e_ref[...],
                   preferred_element_type=f32) + b2e_ref[...]
    x = x_ref[...]
    s_in = x[:, :_NS]
    xv = x[:, _NS:40]
    sh = sh_ref[...]
    sh0 = sh[:, 0:1]
    sh1 = sh[:, 1:4]
    z000 = jnp.dot(s_in, a0_ref[...], preferred_element_type=f32) * sh0
    sh1t = jnp.dot(sh1, tb_ref[...], preferred_element_type=f32)
    z110 = jnp.dot(xv * sh1t, m_ref[...], preferred_element_type=f32)
    z011 = (jnp.dot(s_in, a1_ref[...], preferred_element_type=f32)
            * jnp.dot(sh1, b1m_ref[...], preferred_element_type=f32))
    z101 = jnp.dot(xv, a2_ref[...], preferred_element_type=f32) * sh0
    z = jnp.concatenate([z000, z110, z011, z101], axis=1)
    tp = jnp.dot((wexp * z).astype(jnp.bfloat16), c_ref[...],
                 preferred_element_type=f32)
    o_ref[...] = jnp.concatenate(
        [tp, jnp.zeros((tp.shape[0], 88), f32)], axis=1)


def _tc_tensor_product(edge_feat, x_dst, edge_attr, fc_w1, fc_b1, w2e, b2e,
                       n_edges=_NE, T=2000, interpret=False):
    E_np, C, A0, M, A1, B1, A2, Tb = _build_consts()
    del E_np
    grid = (n_edges // T,)
    row = lambda i: (i, 0)
    rep = lambda i: (0, 0)
    consts = [jnp.asarray(a) for a in (A0, M, A1, B1, A2, Tb)]
    consts.append(jnp.asarray(C, jnp.bfloat16))
    in_specs = (
        [pl.BlockSpec((T, 16), row),
         pl.BlockSpec((T, 128), row),
         pl.BlockSpec((T, 4), row),
         pl.BlockSpec((16, 128), rep),
         pl.BlockSpec((1, 128), rep),
         pl.BlockSpec((128, _NJ), rep),
         pl.BlockSpec((1, _NJ), rep)]
        + [pl.BlockSpec(c.shape, rep) for c in consts]
    )
    return pl.pallas_call(
        _tp_body,
        grid=grid,
        in_specs=in_specs,
        out_specs=pl.BlockSpec((T, 128), row),
        out_shape=jax.ShapeDtypeStruct((n_edges, 128), jnp.float32),
        interpret=interpret,
    )(edge_feat, x_dst, edge_attr, fc_w1, fc_b1[None, :], w2e, b2e[None, :],
      *consts)


_NC = 2
_NSUB = 16
_NW = _NC * _NSUB
_BPW = _NE // _NW
_CHG = 1000
_NCHG = _BPW // _CHG
_CHS = 200
_NCHS = _BPW // _CHS


def _sc_gather(node_attr, dst, n_edges=_NE, chg=_CHG):
    bpw = n_edges // _NW
    nchg = bpw // chg
    mesh = plsc.VectorSubcoreMesh(core_axis_name="c", subcore_axis_name="s")

    @functools.partial(
        pl.kernel,
        out_type=jax.ShapeDtypeStruct((n_edges, 128), jnp.float32),
        mesh=mesh,
        scratch_types=[
            pltpu.VMEM((chg,), jnp.int32),
            pltpu.VMEM((chg, 128), jnp.float32),
            pltpu.SemaphoreType.DMA,
        ],
    )
    def k(table_hbm, idx_hbm, out_hbm, idx_v, rows_v, sem):
        wid = lax.axis_index("s") * _NC + lax.axis_index("c")
        base = wid * bpw
        for c in range(nchg):
            off = base + c * chg
            pltpu.sync_copy(idx_hbm.at[pl.ds(off, chg)], idx_v)
            pltpu.async_copy(table_hbm.at[idx_v], rows_v, sem).wait()
            pltpu.sync_copy(rows_v, out_hbm.at[pl.ds(off, chg)])

    return k(node_attr, dst)


_SCB = 128
_EPT = _NE // _NSUB
_NSCB = _EPT // _SCB
_STAIL = _EPT - _NSCB * _SCB
_HALF = _NN // _NC
_ACCR = _HALF + 120


def _sc_scatter(tp128, src, zeros, n_edges=_NE):
    ept = n_edges // _NSUB
    nscb = ept // _SCB
    stail = ept - nscb * _SCB
    pairs = nscb // 2
    leftover = nscb - 2 * pairs
    tsz = max(stail, 8)
    mesh = plsc.VectorSubcoreMesh(core_axis_name="c", subcore_axis_name="s")

    @functools.partial(
        pl.kernel,
        out_type=jax.ShapeDtypeStruct((_NN, 128), jnp.float32),
        mesh=mesh,
        scratch_types=[
            pltpu.VMEM((_SCB,), jnp.int32),
            pltpu.VMEM((_SCB,), jnp.int32),
            pltpu.VMEM((_SCB, 128), jnp.float32),
            pltpu.VMEM((_SCB,), jnp.int32),
            pltpu.VMEM((_SCB,), jnp.int32),
            pltpu.VMEM((_SCB, 128), jnp.float32),
            pltpu.VMEM((tsz,), jnp.int32),
            pltpu.VMEM((tsz,), jnp.int32),
            pltpu.VMEM((tsz, 128), jnp.float32),
            pltpu.VMEM_SHARED((_ACCR, 128), jnp.float32),
            pltpu.SemaphoreType.DMA,
            pltpu.SemaphoreType.DMA,
        ],
    )
    def k(tp_hbm, src_hbm, z_hbm, out_hbm,
          idx_v, lidx_v, rows_v, idx_v2, lidx_v2, rows_v2,
          idx_t, lidx_t, rows_t, acc_sh, sem, sem2):
        cid = lax.axis_index("c")
        sid = lax.axis_index("s")
        lo = cid * _HALF
        zpt = _ACCR // _NSUB
        pltpu.sync_copy(z_hbm.at[pl.ds(sid * zpt, zpt)],
                        acc_sh.at[pl.ds(sid * zpt, zpt)])
        plsc.subcore_barrier()

        def remap(n, src_idx, dst_idx):
            for g in range(n // 16):
                v = src_idx[pl.ds(g * 16, 16)]
                lv = v - lo
                ok = (lv >= 0) & (lv < _HALF)
                dst_idx[pl.ds(g * 16, 16)] = jnp.where(ok, lv, _HALF)

        base = sid * ept
        def fetch(j, b):
            iv, rv, sm = (idx_v, rows_v, sem) if b == 0 else (idx_v2, rows_v2, sem2)
            off = base + j * _SCB
            pltpu.async_copy(src_hbm.at[pl.ds(off, _SCB)], iv, sm)
            pltpu.async_copy(tp_hbm.at[pl.ds(off, _SCB)], rv, sm)

        def drain(b):
            iv, rv, sm = (idx_v, rows_v, sem) if b == 0 else (idx_v2, rows_v2, sem2)
            pltpu.make_async_copy(src_hbm.at[pl.ds(0, _SCB)], iv, sm).wait()
            pltpu.make_async_copy(tp_hbm.at[pl.ds(0, _SCB)], rv, sm).wait()

        def scat(b):
            iv, rv, _ = (idx_v, rows_v, sem) if b == 0 else (idx_v2, rows_v2, sem2)
            li = lidx_v if b == 0 else lidx_v2
            remap(_SCB, iv, li)
            pltpu.sync_copy(rv, acc_sh.at[li], add=True)

        fetch(0, 0)
        def body(i, carry):
            fetch(2 * i + 1, 1)
            drain(0)
            scat(0)
            @pl.when(i < pairs - 1)
            def _():
                fetch(2 * i + 2, 0)
            drain(1)
            scat(1)
            return carry
        lax.fori_loop(0, pairs, body, 0)
        if leftover:
            offl = base + 2 * pairs * _SCB
            pltpu.sync_copy(src_hbm.at[pl.ds(offl, _SCB)], idx_v)
            pltpu.sync_copy(tp_hbm.at[pl.ds(offl, _SCB)], rows_v)
            remap(_SCB, idx_v, lidx_v)
            pltpu.sync_copy(rows_v, acc_sh.at[lidx_v], add=True)
        if stail:
            offt = base + nscb * _SCB
            pltpu.sync_copy(src_hbm.at[pl.ds(offt, stail)], idx_t)
            pltpu.sync_copy(tp_hbm.at[pl.ds(offt, stail)], rows_t)
            remap(stail, idx_t, lidx_t)
            pltpu.sync_copy(rows_t, acc_sh.at[lidx_t], add=True)
        plsc.subcore_barrier()
        @pl.when(sid < 5)
        def _():
            pltpu.sync_copy(acc_sh.at[pl.ds(sid * 1000, 1000)],
                            out_hbm.at[pl.ds(cid * _HALF + sid * 1000, 1000)])

    return k(tp128, src, zeros)


def _comb_body(a_ref, b_ref, o_ref):
    o_ref[...] = a_ref[:, :40] + b_ref[:, :40]


def _tc_combine(a, b):
    tn = 2000
    return pl.pallas_call(
        _comb_body,
        grid=(_NN // tn,),
        in_specs=[pl.BlockSpec((tn, 128), lambda i: (i, 0)),
                  pl.BlockSpec((tn, 128), lambda i: (i, 0))],
        out_specs=pl.BlockSpec((tn, 40), lambda i: (i, 0)),
        out_shape=jax.ShapeDtypeStruct((_NN, 40), jnp.float32),
    )(a, b)


_SLABS = ((0, 96000, 600, 2000), (96000, 64000, 1000, 2000))


def kernel(node_attr, edge_index, edge_attr, edge_feat,
           fc_w1, fc_b1, fc_w2, fc_b2):
    E_np = _build_consts()[0]
    Ej = jnp.asarray(E_np)
    w2e = (fc_w2 @ Ej).astype(jnp.bfloat16)
    b2e = fc_b2 @ Ej
    dst = edge_index[1]
    src = edge_index[0]
    node_pad = jnp.pad(node_attr, ((0, 0), (0, 128 - 40)))
    zeros = jnp.zeros((_ACCR, 128), jnp.float32)
    outs = []
    for off, S, chg, T in _SLABS:
        x = _sc_gather(node_pad, lax.dynamic_slice_in_dim(dst, off, S), S, chg)
        tp = _tc_tensor_product(
            lax.dynamic_slice_in_dim(edge_feat, off, S), x,
            lax.dynamic_slice_in_dim(edge_attr, off, S),
            fc_w1, fc_b1, w2e, b2e, n_edges=S, T=T)
        outs.append(_sc_scatter(
            tp, lax.dynamic_slice_in_dim(src, off, S), zeros, n_edges=S))
    return _tc_combine(outs[0], outs[1])


# speedup vs baseline: 4.4697x; 1.0272x over previous
import functools

import numpy as np
import jax
import jax.numpy as jnp
from jax import lax
from jax.experimental import pallas as pl
from jax.experimental.pallas import tpu as pltpu
from jax.experimental.pallas import tpu_sc as plsc

_NS = 16
_NV = 8
_NE = 160000
_NN = 10000
_NJ = 960
_ALPHA = 1.0 / np.sqrt(24.0)


@functools.cache
def _build_consts():
    E = np.zeros((576, _NJ), np.float32)
    C = np.zeros((_NJ, 40), np.float32)
    A0 = np.zeros((16, 256), np.float32)
    M = np.zeros((24, 128), np.float32)
    A1 = np.zeros((16, 384), np.float32)
    B1 = np.zeros((3, 384), np.float32)
    A2 = np.zeros((24, 192), np.float32)
    Tb = np.zeros((3, 24), np.float32)
    s3 = 1.0 / np.sqrt(3.0)
    for u in range(16):
        for w in range(16):
            j = u * 16 + w
            E[u * 16 + w, j] = 1.0
            C[j, w] = _ALPHA
            A0[u, j] = 1.0
    for u in range(8):
        for w in range(16):
            j = 256 + u * 16 + w
            E[448 + u * 16 + w, j] = 1.0
            C[j, w] = _ALPHA * s3
        for i in range(3):
            Tb[i, u * 3 + i] = 1.0
            for w in range(16):
                M[u * 3 + i, u * 16 + w] = 1.0
    for u in range(16):
        for w in range(8):
            for i in range(3):
                j = 384 + (u * 8 + w) * 3 + i
                E[256 + u * 8 + w, j] = 1.0
                C[j, 16 + w * 3 + i] = _ALPHA
                A1[u, j - 384] = 1.0
                B1[i, j - 384] = 1.0
    for u in range(8):
        for w in range(8):
            for i in range(3):
                j = 768 + (u * 8 + w) * 3 + i
                E[384 + u * 8 + w, j] = 1.0
                C[j, 16 + w * 3 + i] = _ALPHA
                A2[u * 3 + i, j - 768] = 1.0
    return E, C, A0, M, A1, B1, A2, Tb


def _tp_body(ef_ref, x_ref, sh_ref, w1_ref, b1_ref, w2e_ref, b2e_ref,
             a0_ref, m_ref, a1_ref, b1m_ref, a2_ref, tb_ref, c_ref, o_ref):
    f32 = jnp.float32
    h = jnp.maximum(
        jnp.dot(ef_ref[...], w1_ref[...], preferred_element_type=f32)
        + b1_ref[...], 0.0)
    wexp = jnp.dot(h.astype(jnp.bfloat16), w2e_ref[...],
                   preferred_element_type=f32) + b2e_ref[...]
    x = x_ref[...]
    s_in = x[:, :_NS]
    xv = x[:, _NS:40]
    sh = sh_ref[...]
    sh0 = sh[:, 0:1]
    sh1 = sh[:, 1:4]
    z000 = jnp.dot(s_in, a0_ref[...], preferred_element_type=f32) * sh0
    sh1t = jnp.dot(sh1, tb_ref[...], preferred_element_type=f32)
    z110 = jnp.dot(xv * sh1t, m_ref[...], preferred_element_type=f32)
    z011 = (jnp.dot(s_in, a1_ref[...], preferred_element_type=f32)
            * jnp.dot(sh1, b1m_ref[...], preferred_element_type=f32))
    z101 = jnp.dot(xv, a2_ref[...], preferred_element_type=f32) * sh0
    z = jnp.concatenate([z000, z110, z011, z101], axis=1)
    tp = jnp.dot((wexp * z).astype(jnp.bfloat16), c_ref[...],
                 preferred_element_type=f32)
    o_ref[...] = jnp.concatenate(
        [tp, jnp.zeros((tp.shape[0], 88), f32)], axis=1)


def _tc_tensor_product(edge_feat, x_dst, edge_attr, fc_w1, fc_b1, w2e, b2e,
                       n_edges=_NE, T=2000, interpret=False):
    E_np, C, A0, M, A1, B1, A2, Tb = _build_consts()
    del E_np
    grid = (n_edges // T,)
    row = lambda i: (i, 0)
    rep = lambda i: (0, 0)
    consts = [jnp.asarray(a) for a in (A0, M, A1, B1, A2, Tb)]
    consts.append(jnp.asarray(C, jnp.bfloat16))
    in_specs = (
        [pl.BlockSpec((T, 16), row),
         pl.BlockSpec((T, 128), row),
         pl.BlockSpec((T, 4), row),
         pl.BlockSpec((16, 128), rep),
         pl.BlockSpec((1, 128), rep),
         pl.BlockSpec((128, _NJ), rep),
         pl.BlockSpec((1, _NJ), rep)]
        + [pl.BlockSpec(c.shape, rep) for c in consts]
    )
    return pl.pallas_call(
        _tp_body,
        grid=grid,
        in_specs=in_specs,
        out_specs=pl.BlockSpec((T, 128), row),
        out_shape=jax.ShapeDtypeStruct((n_edges, 128), jnp.float32),
        interpret=interpret,
    )(edge_feat, x_dst, edge_attr, fc_w1, fc_b1[None, :], w2e, b2e[None, :],
      *consts)


_NC = 2
_NSUB = 16
_NW = _NC * _NSUB
_BPW = _NE // _NW
_CHG = 1000
_NCHG = _BPW // _CHG
_CHS = 200
_NCHS = _BPW // _CHS


def _sc_gather(node_attr, dst, n_edges=_NE, chg=_CHG):
    bpw = n_edges // _NW
    nchg = bpw // chg
    mesh = plsc.VectorSubcoreMesh(core_axis_name="c", subcore_axis_name="s")

    @functools.partial(
        pl.kernel,
        out_type=jax.ShapeDtypeStruct((n_edges, 128), jnp.float32),
        mesh=mesh,
        scratch_types=[
            pltpu.VMEM((chg,), jnp.int32),
            pltpu.VMEM((chg, 128), jnp.float32),
            pltpu.SemaphoreType.DMA,
        ],
    )
    def k(table_hbm, idx_hbm, out_hbm, idx_v, rows_v, sem):
        wid = lax.axis_index("s") * _NC + lax.axis_index("c")
        base = wid * bpw
        for c in range(nchg):
            off = base + c * chg
            pltpu.sync_copy(idx_hbm.at[pl.ds(off, chg)], idx_v)
            pltpu.async_copy(table_hbm.at[idx_v], rows_v, sem).wait()
            pltpu.sync_copy(rows_v, out_hbm.at[pl.ds(off, chg)])

    return k(node_attr, dst)


_SCB = 128
_EPT = _NE // _NSUB
_NSCB = _EPT // _SCB
_STAIL = _EPT - _NSCB * _SCB
_HALF = _NN // _NC
_ACCR = _HALF + 120


def _sc_scatter(tp128, src, zeros, n_edges=_NE):
    ept = n_edges // _NSUB
    nscb = ept // _SCB
    stail = ept - nscb * _SCB
    pairs = nscb // 2
    leftover = nscb - 2 * pairs
    tsz = max(stail, 8)
    mesh = plsc.VectorSubcoreMesh(core_axis_name="c", subcore_axis_name="s")

    @functools.partial(
        pl.kernel,
        out_type=jax.ShapeDtypeStruct((_NN, 128), jnp.float32),
        mesh=mesh,
        scratch_types=[
            pltpu.VMEM((_SCB,), jnp.int32),
            pltpu.VMEM((_SCB,), jnp.int32),
            pltpu.VMEM((_SCB, 128), jnp.float32),
            pltpu.VMEM((_SCB,), jnp.int32),
            pltpu.VMEM((_SCB,), jnp.int32),
            pltpu.VMEM((_SCB, 128), jnp.float32),
            pltpu.VMEM((tsz,), jnp.int32),
            pltpu.VMEM((tsz,), jnp.int32),
            pltpu.VMEM((tsz, 128), jnp.float32),
            pltpu.VMEM_SHARED((_ACCR, 128), jnp.float32),
            pltpu.SemaphoreType.DMA,
            pltpu.SemaphoreType.DMA,
        ],
    )
    def k(tp_hbm, src_hbm, z_hbm, out_hbm,
          idx_v, lidx_v, rows_v, idx_v2, lidx_v2, rows_v2,
          idx_t, lidx_t, rows_t, acc_sh, sem, sem2):
        cid = lax.axis_index("c")
        sid = lax.axis_index("s")
        lo = cid * _HALF
        zpt = _ACCR // _NSUB
        pltpu.sync_copy(z_hbm.at[pl.ds(sid * zpt, zpt)],
                        acc_sh.at[pl.ds(sid * zpt, zpt)])
        plsc.subcore_barrier()

        def remap(n, src_idx, dst_idx):
            for g in range(n // 16):
                v = src_idx[pl.ds(g * 16, 16)]
                lv = v - lo
                ok = (lv >= 0) & (lv < _HALF)
                dst_idx[pl.ds(g * 16, 16)] = jnp.where(ok, lv, _HALF)

        base = sid * ept
        def fetch(j, b):
            iv, rv, sm = (idx_v, rows_v, sem) if b == 0 else (idx_v2, rows_v2, sem2)
            off = base + j * _SCB
            pltpu.async_copy(src_hbm.at[pl.ds(off, _SCB)], iv, sm)
            pltpu.async_copy(tp_hbm.at[pl.ds(off, _SCB)], rv, sm)

        def drain(b):
            iv, rv, sm = (idx_v, rows_v, sem) if b == 0 else (idx_v2, rows_v2, sem2)
            pltpu.make_async_copy(src_hbm.at[pl.ds(0, _SCB)], iv, sm).wait()
            pltpu.make_async_copy(tp_hbm.at[pl.ds(0, _SCB)], rv, sm).wait()

        def scat(b):
            iv, rv, _ = (idx_v, rows_v, sem) if b == 0 else (idx_v2, rows_v2, sem2)
            li = lidx_v if b == 0 else lidx_v2
            remap(_SCB, iv, li)
            pltpu.sync_copy(rv, acc_sh.at[li], add=True)

        fetch(0, 0)
        def body(i, carry):
            fetch(2 * i + 1, 1)
            drain(0)
            scat(0)
            @pl.when(i < pairs - 1)
            def _():
                fetch(2 * i + 2, 0)
            drain(1)
            scat(1)
            return carry
        lax.fori_loop(0, pairs, body, 0)
        if leftover:
            offl = base + 2 * pairs * _SCB
            pltpu.sync_copy(src_hbm.at[pl.ds(offl, _SCB)], idx_v)
            pltpu.sync_copy(tp_hbm.at[pl.ds(offl, _SCB)], rows_v)
            remap(_SCB, idx_v, lidx_v)
            pltpu.sync_copy(rows_v, acc_sh.at[lidx_v], add=True)
        if stail:
            offt = base + nscb * _SCB
            pltpu.sync_copy(src_hbm.at[pl.ds(offt, stail)], idx_t)
            pltpu.sync_copy(tp_hbm.at[pl.ds(offt, stail)], rows_t)
            remap(stail, idx_t, lidx_t)
            pltpu.sync_copy(rows_t, acc_sh.at[lidx_t], add=True)
        plsc.subcore_barrier()
        @pl.when(sid < 5)
        def _():
            pltpu.sync_copy(acc_sh.at[pl.ds(sid * 1000, 1000)],
                            out_hbm.at[pl.ds(cid * _HALF + sid * 1000, 1000)])

    return k(tp128, src, zeros)


def _comb_body(a_ref, b_ref, o_ref):
    o_ref[...] = a_ref[:, :40] + b_ref[:, :40]


def _tc_combine(a, b):
    tn = 2000
    return pl.pallas_call(
        _comb_body,
        grid=(_NN // tn,),
        in_specs=[pl.BlockSpec((tn, 128), lambda i: (i, 0)),
                  pl.BlockSpec((tn, 128), lambda i: (i, 0))],
        out_specs=pl.BlockSpec((tn, 40), lambda i: (i, 0)),
        out_shape=jax.ShapeDtypeStruct((_NN, 40), jnp.float32),
    )(a, b)


_SLABS = ((0, 96000, 600, 4000), (96000, 64000, 1000, 4000))


def kernel(node_attr, edge_index, edge_attr, edge_feat,
           fc_w1, fc_b1, fc_w2, fc_b2):
    E_np = _build_consts()[0]
    Ej = jnp.asarray(E_np)
    w2e = (fc_w2 @ Ej).astype(jnp.bfloat16)
    b2e = fc_b2 @ Ej
    dst = edge_index[1]
    src = edge_index[0]
    node_pad = jnp.pad(node_attr, ((0, 0), (0, 128 - 40)))
    zeros = jnp.zeros((_ACCR, 128), jnp.float32)
    outs = []
    for off, S, chg, T in _SLABS:
        x = _sc_gather(node_pad, lax.dynamic_slice_in_dim(dst, off, S), S, chg)
        tp = _tc_tensor_product(
            lax.dynamic_slice_in_dim(edge_feat, off, S), x,
            lax.dynamic_slice_in_dim(edge_attr, off, S),
            fc_w1, fc_b1, w2e, b2e, n_edges=S, T=T)
        outs.append(_sc_scatter(
            tp, lax.dynamic_slice_in_dim(src, off, S), zeros, n_edges=S))
    return _tc_combine(outs[0], outs[1])
